# Initial kernel scaffold; baseline (speedup 1.0000x reference)
import jax
import jax.numpy as jnp
from jax.experimental import pallas as pl


def kernel(x, edge_index, W1, b1, W2, b2, W3, b3):
    raise NotImplementedError("write your pallas kernel here")


# speedup vs baseline: 12.4238x; 12.4238x over previous
import functools

import jax
import jax.numpy as jnp
from jax import lax
from jax.experimental import pallas as pl
from jax.experimental.pallas import tpu as pltpu
from jax.experimental.pallas import tpu_sc as plsc

N = 10000
E = 320000
D = 128
K = 10
ALPHA = 0.1

NC = 2
NS = 16
NW = NC * NS
EPW = E // NW
CH = 80
NCH = EPW // CH
RPT = N // NS
NP = 10240
RPTP = NP // NS

_mesh = plsc.VectorSubcoreMesh(core_axis_name="c", subcore_axis_name="s")


@functools.partial(
    pl.kernel,
    out_type=jax.ShapeDtypeStruct((NC, NP), jnp.float32),
    mesh=_mesh,
    scratch_types=[
        pltpu.VMEM((NCH, CH), jnp.int32),
        pltpu.VMEM((CH,), jnp.float32),
        pltpu.VMEM((RPTP,), jnp.float32),
        pltpu.VMEM_SHARED((NP,), jnp.float32),
    ],
)
def _deg(col_hbm, out_hbm, col_v, ones_v, zbuf_v, deg_sh):
    c = lax.axis_index("c")
    s = lax.axis_index("s")
    wid = c * NS + s
    pltpu.sync_copy(col_hbm.at[wid], col_v)
    for i in range(CH // 16):
        ones_v[pl.ds(i * 16, 16)] = jnp.full((16,), 1.0, jnp.float32)

    def zb(i, carry):
        zbuf_v[pl.ds(i * 16, 16)] = jnp.zeros((16,), jnp.float32)
        return carry

    lax.fori_loop(0, RPTP // 16, zb, 0)
    pltpu.sync_copy(zbuf_v, deg_sh.at[pl.ds(s * RPTP, RPTP)])
    plsc.subcore_barrier()

    def body(j, carry):
        pltpu.sync_copy(ones_v, deg_sh.at[col_v.at[j]], add=True)
        return carry

    lax.fori_loop(0, NCH, body, 0)
    plsc.subcore_barrier()
    pltpu.sync_copy(deg_sh.at[pl.ds(s * RPTP, RPTP)],
                    out_hbm.at[c, pl.ds(s * RPTP, RPTP)])


@functools.partial(
    pl.kernel,
    out_type=jax.ShapeDtypeStruct((NC, NP, D), jnp.float32),
    mesh=_mesh,
    scratch_types=[
        pltpu.VMEM((NCH, CH), jnp.int32),
        pltpu.VMEM((NCH, CH), jnp.int32),
        pltpu.VMEM((CH, D), jnp.float32),
        pltpu.VMEM_SHARED((NP, D), jnp.float32),
        pltpu.SemaphoreType.DMA,
    ],
)
def _step(zhat_hbm, row_hbm, col_hbm, out_hbm, row_v, col_v, buf, agg_sh, gsem):
    c = lax.axis_index("c")
    s = lax.axis_index("s")
    wid = c * NS + s
    pltpu.sync_copy(row_hbm.at[wid], row_v)
    pltpu.sync_copy(col_hbm.at[wid], col_v)

    def zb(i, carry):
        for v in range(D // 16):
            buf[i, pl.ds(v * 16, 16)] = jnp.zeros((16,), jnp.float32)
        return carry

    lax.fori_loop(0, CH, zb, 0)
    nzc = NP // CH // NS
    lo = s * nzc

    def zc(i, carry):
        pltpu.sync_copy(buf, agg_sh.at[pl.ds(i * CH, CH)])
        return carry

    lax.fori_loop(lo, lo + nzc, zc, 0)
    plsc.subcore_barrier()

    def body(j, carry):
        pltpu.async_copy(zhat_hbm.at[row_v.at[j]], buf, gsem).wait()
        pltpu.sync_copy(buf, agg_sh.at[col_v.at[j]], add=True)
        return carry

    lax.fori_loop(0, NCH, body, 0)
    plsc.subcore_barrier()
    pltpu.sync_copy(agg_sh.at[pl.ds(s * RPTP, RPTP)],
                    out_hbm.at[c, pl.ds(s * RPTP, RPTP)])


def _prep_body(x_ref, w1_ref, b1_ref, w2_ref, b2_ref, degp_ref,
               h_ref, dis_ref, rdis_ref):
    h1 = jnp.maximum(
        jnp.dot(x_ref[...], w1_ref[...], preferred_element_type=jnp.float32)
        + b1_ref[...], 0.0)
    h_ref[...] = (
        jnp.dot(h1, w2_ref[...], preferred_element_type=jnp.float32)
        + b2_ref[...])
    deg = jnp.sum(degp_ref[...], axis=0, keepdims=True) + 1.0
    dis_ref[...] = lax.rsqrt(deg)
    rdis_ref[...] = jnp.sqrt(deg)


def _scale_body(h_ref, dis_ref, o_ref):
    o_ref[...] = h_ref[...] * dis_ref[...]


def _combine_body(agg_ref, zhat_ref, h_ref, dis_ref, zhat_out):
    ssum = agg_ref[0] + agg_ref[1] + zhat_ref[...]
    zn = (1.0 - ALPHA) * (dis_ref[...] * ssum) + ALPHA * h_ref[...]
    zhat_out[...] = dis_ref[...] * zn


def _final_body(zhat_ref, rdis_ref, w3_ref, b3_ref, o_ref):
    z = zhat_ref[...] * rdis_ref[...]
    o_ref[...] = (
        jnp.dot(z, w3_ref[...], preferred_element_type=jnp.float32)
        + b3_ref[...])


def kernel(x, edge_index, W1, b1, W2, b2, W3, b3):
    row = edge_index[0].reshape(NW, NCH, CH)
    col = edge_index[1].reshape(NW, NCH, CH)

    degp = _deg(col)

    h, dis_row, rdis_row = pl.pallas_call(
        _prep_body,
        out_shape=[
            jax.ShapeDtypeStruct((N, D), jnp.float32),
            jax.ShapeDtypeStruct((1, NP), jnp.float32),
            jax.ShapeDtypeStruct((1, NP), jnp.float32),
        ],
    )(x, W1, b1.reshape(1, D), W2, b2.reshape(1, D), degp)
    dis_col = dis_row.reshape(NP, 1)
    rdis_col = rdis_row.reshape(NP, 1)

    h_p = jnp.pad(h, ((0, NP - N), (0, 0)))

    zhat = pl.pallas_call(
        _scale_body,
        out_shape=jax.ShapeDtypeStruct((NP, D), jnp.float32),
    )(h_p, dis_col)

    combine = pl.pallas_call(
        _combine_body,
        out_shape=jax.ShapeDtypeStruct((NP, D), jnp.float32),
    )
    for _ in range(K):
        agg = _step(zhat, row, col)
        zhat = combine(agg, zhat, h_p, dis_col)

    out = pl.pallas_call(
        _final_body,
        out_shape=jax.ShapeDtypeStruct((NP, D), jnp.float32),
    )(zhat, rdis_col, W3, b3.reshape(1, D))
    return out[:N]
